# baseline (speedup 1.0000x reference)
import functools

import jax
import jax.numpy as jnp
from jax import lax
from jax.experimental import pallas as pl
from jax.experimental.pallas import tpu as pltpu
from jax.experimental.pallas import tpu_sc as plsc

BATCH = 16384
EMB = 32
NC = 2
NS = 16
NW = NC * NS
BPW = BATCH // NW

_sc_mesh = plsc.VectorSubcoreMesh(core_axis_name="c", subcore_axis_name="s")


@functools.partial(
    pl.kernel,
    out_type=(
        jax.ShapeDtypeStruct((BATCH, EMB), jnp.float32),
        jax.ShapeDtypeStruct((BATCH, EMB), jnp.float32),
    ),
    mesh=_sc_mesh,
    scratch_types=[
        pltpu.SMEM((BPW,), jnp.int32),
        pltpu.SMEM((BPW,), jnp.int32),
        pltpu.VMEM_SHARED((NS, 2 * BPW), jnp.int32),
        pltpu.SemaphoreType.DMA,
    ],
)
def _sc_gather(u_hbm, v_hbm, uemb_hbm, vemb_hbm, ue_hbm, ve_hbm,
               uidx, vidx, idx_stage, sem):
    wid = lax.axis_index("s") * NC + lax.axis_index("c")
    sid = lax.axis_index("s")
    base = wid * BPW
    pltpu.sync_copy(u_hbm.at[wid], idx_stage.at[sid, pl.ds(0, BPW)])
    pltpu.sync_copy(v_hbm.at[wid], idx_stage.at[sid, pl.ds(BPW, BPW)])
    pltpu.sync_copy(idx_stage.at[sid, pl.ds(0, BPW)], uidx)
    pltpu.sync_copy(idx_stage.at[sid, pl.ds(BPW, BPW)], vidx)

    def issue(k, c):
        pltpu.async_copy(uemb_hbm.at[uidx[k]], ue_hbm.at[base + k], sem)
        pltpu.async_copy(vemb_hbm.at[vidx[k]], ve_hbm.at[base + k], sem)
        return c

    lax.fori_loop(0, BPW, issue, 0)

    def drain(k, c):
        pltpu.make_async_copy(uemb_hbm.at[0], ue_hbm.at[0], sem).wait()
        pltpu.make_async_copy(vemb_hbm.at[0], ve_hbm.at[0], sem).wait()
        return c

    lax.fori_loop(0, BPW, drain, 0)


MLP_BLK = 2048


def _mlp_body(ue, ve, w1a, w1b, b1, w2, b2, w3, b3, w4, b4, w5, b5, out):
    x = jnp.maximum(ue[...] @ w1a[...] + ve[...] @ w1b[...] + b1[...], 0.0)
    x = jnp.maximum(x @ w2[...] + b2[...], 0.0)
    x = jnp.maximum(x @ w3[...] + b3[...], 0.0)
    x = jnp.maximum(x @ w4[...] + b4[...], 0.0)
    out[...] = x @ w5[...] + b5[...]


def _full(shape):
    return pl.BlockSpec(shape, lambda i: (0, 0))


_mlp_call = pl.pallas_call(
    _mlp_body,
    grid=(BATCH // MLP_BLK,),
    in_specs=[
        pl.BlockSpec((MLP_BLK, EMB), lambda i: (i, 0)),
        pl.BlockSpec((MLP_BLK, EMB), lambda i: (i, 0)),
        _full((EMB, 64)), _full((EMB, 64)), _full((1, 64)),
        _full((64, 32)), _full((1, 32)),
        _full((32, 16)), _full((1, 16)),
        _full((16, 8)), _full((1, 8)),
        _full((8, 1)), _full((1, 1)),
    ],
    out_specs=pl.BlockSpec((MLP_BLK, 1), lambda i: (i, 0)),
    out_shape=jax.ShapeDtypeStruct((BATCH, 1), jnp.float32),
)


def kernel(u, v, user_emb, item_emb, W1, b1, W2, b2, W3, b3, W4, b4, W5, b5):
    u = u.astype(jnp.int32).reshape(NW, BPW)
    v = v.astype(jnp.int32).reshape(NW, BPW)
    ue, ve = _sc_gather(u, v, user_emb, item_emb)
    out = _mlp_call(
        ue, ve,
        W1[:EMB], W1[EMB:], b1.reshape(1, 64),
        W2, b2.reshape(1, 32),
        W3, b3.reshape(1, 16),
        W4, b4.reshape(1, 8),
        W5, b5.reshape(1, 1),
    )
    return jnp.squeeze(out, axis=1)


# speedup vs baseline: 1.6265x; 1.6265x over previous
import functools

import jax
import jax.numpy as jnp
from jax import lax
from jax.experimental import pallas as pl
from jax.experimental.pallas import tpu as pltpu
from jax.experimental.pallas import tpu_sc as plsc

BATCH = 16384
EMB = 32
TILE_H = 8
NC = 2
NS = 16
NW = NC * NS
BPW = BATCH // NW
G = 32
NCH = BPW // G

_sc_mesh = plsc.VectorSubcoreMesh(core_axis_name="c", subcore_axis_name="s")


@functools.partial(
    pl.kernel,
    out_type=(
        jax.ShapeDtypeStruct((BATCH * EMB,), jnp.float32),
        jax.ShapeDtypeStruct((BATCH * EMB,), jnp.float32),
    ),
    mesh=_sc_mesh,
    scratch_types=[
        pltpu.SMEM((BPW,), jnp.int32),
        pltpu.SMEM((BPW,), jnp.int32),
        pltpu.VMEM_SHARED((NS, 2 * BPW), jnp.int32),
        pltpu.VMEM((G, TILE_H, EMB), jnp.float32),
        pltpu.VMEM((G, TILE_H, EMB), jnp.float32),
        pltpu.VMEM((G * EMB,), jnp.float32),
        pltpu.VMEM((G * EMB,), jnp.float32),
        pltpu.SemaphoreType.DMA,
    ],
)
def _sc_gather(u_hbm, v_hbm, uemb_hbm, vemb_hbm, ue_hbm, ve_hbm,
               uidx, vidx, idx_stage, tbu, tbv, rbu, rbv, sem):
    wid = lax.axis_index("s") * NC + lax.axis_index("c")
    sid = lax.axis_index("s")
    base = wid * BPW
    pltpu.sync_copy(u_hbm.at[wid], idx_stage.at[sid, pl.ds(0, BPW)])
    pltpu.sync_copy(v_hbm.at[wid], idx_stage.at[sid, pl.ds(BPW, BPW)])
    pltpu.sync_copy(idx_stage.at[sid, pl.ds(0, BPW)], uidx)
    pltpu.sync_copy(idx_stage.at[sid, pl.ds(BPW, BPW)], vidx)

    def chunk_body(ch, carry):
        off = ch * G

        def issue(k, c):
            gu = pl.multiple_of(lax.bitwise_and(uidx[off + k], ~(TILE_H - 1)), TILE_H)
            pltpu.async_copy(uemb_hbm.at[pl.ds(gu, TILE_H)], tbu.at[k], sem)
            gv = pl.multiple_of(lax.bitwise_and(vidx[off + k], ~(TILE_H - 1)), TILE_H)
            pltpu.async_copy(vemb_hbm.at[pl.ds(gv, TILE_H)], tbv.at[k], sem)
            return c

        lax.fori_loop(0, G, issue, 0)

        def drain(k, c):
            pltpu.make_async_copy(uemb_hbm.at[pl.ds(0, TILE_H)], tbu.at[0], sem).wait()
            pltpu.make_async_copy(vemb_hbm.at[pl.ds(0, TILE_H)], tbv.at[0], sem).wait()
            return c

        lax.fori_loop(0, G, drain, 0)

        for k in range(G):
            ru = lax.bitwise_and(uidx[off + k], TILE_H - 1)
            rv = lax.bitwise_and(vidx[off + k], TILE_H - 1)
            for h in range(EMB // 16):
                rbu[pl.ds(k * EMB + h * 16, 16)] = tbu[k, ru, pl.ds(h * 16, 16)]
                rbv[pl.ds(k * EMB + h * 16, 16)] = tbv[k, rv, pl.ds(h * 16, 16)]

        pltpu.sync_copy(rbu, ue_hbm.at[pl.ds((base + off) * EMB, G * EMB)])
        pltpu.sync_copy(rbv, ve_hbm.at[pl.ds((base + off) * EMB, G * EMB)])
        return carry

    lax.fori_loop(0, NCH, chunk_body, 0)


MLP_BLK = 2048


def _mlp_body(ue, ve, w1a, w1b, b1, w2, b2, w3, b3, w4, b4, w5, b5, out):
    x = jnp.maximum(ue[...] @ w1a[...] + ve[...] @ w1b[...] + b1[...], 0.0)
    x = jnp.maximum(x @ w2[...] + b2[...], 0.0)
    x = jnp.maximum(x @ w3[...] + b3[...], 0.0)
    x = jnp.maximum(x @ w4[...] + b4[...], 0.0)
    out[...] = x @ w5[...] + b5[...]


def _full(shape):
    return pl.BlockSpec(shape, lambda i: (0, 0))


_mlp_call = pl.pallas_call(
    _mlp_body,
    grid=(BATCH // MLP_BLK,),
    in_specs=[
        pl.BlockSpec((MLP_BLK, EMB), lambda i: (i, 0)),
        pl.BlockSpec((MLP_BLK, EMB), lambda i: (i, 0)),
        _full((EMB, 64)), _full((EMB, 64)), _full((1, 64)),
        _full((64, 32)), _full((1, 32)),
        _full((32, 16)), _full((1, 16)),
        _full((16, 8)), _full((1, 8)),
        _full((8, 1)), _full((1, 1)),
    ],
    out_specs=pl.BlockSpec((MLP_BLK, 1), lambda i: (i, 0)),
    out_shape=jax.ShapeDtypeStruct((BATCH, 1), jnp.float32),
)


def kernel(u, v, user_emb, item_emb, W1, b1, W2, b2, W3, b3, W4, b4, W5, b5):
    u = u.astype(jnp.int32).reshape(NW, BPW)
    v = v.astype(jnp.int32).reshape(NW, BPW)
    ue, ve = _sc_gather(u, v, user_emb, item_emb)
    ue = ue.reshape(BATCH, EMB)
    ve = ve.reshape(BATCH, EMB)
    out = _mlp_call(
        ue, ve,
        W1[:EMB], W1[EMB:], b1.reshape(1, 64),
        W2, b2.reshape(1, 32),
        W3, b3.reshape(1, 16),
        W4, b4.reshape(1, 8),
        W5, b5.reshape(1, 1),
    )
    return jnp.squeeze(out, axis=1)


# speedup vs baseline: 3.0197x; 1.8566x over previous
import functools

import jax
import jax.numpy as jnp
from jax import lax
from jax.experimental import pallas as pl
from jax.experimental.pallas import tpu as pltpu
from jax.experimental.pallas import tpu_sc as plsc

BATCH = 16384
EMB = 32
TILE_H = 8
NGRP = 1000000 // TILE_H
NC = 2
NS = 16
NW = NC * NS
BPW = BATCH // NW
CH = 128
NCH = BPW // CH

_sc_mesh = plsc.VectorSubcoreMesh(core_axis_name="c", subcore_axis_name="s")


@functools.partial(
    pl.kernel,
    out_type=(
        jax.ShapeDtypeStruct((BATCH, EMB), jnp.float32),
        jax.ShapeDtypeStruct((BATCH, EMB), jnp.float32),
    ),
    mesh=_sc_mesh,
    scratch_types=[
        pltpu.SMEM((BPW,), jnp.int32),
        pltpu.SMEM((BPW,), jnp.int32),
        pltpu.VMEM_SHARED((NS, 2 * BPW), jnp.int32),
        pltpu.VMEM((CH, EMB), jnp.float32),
        pltpu.VMEM((CH, EMB), jnp.float32),
        pltpu.SemaphoreType.DMA,
    ],
)
def _sc_gather(u_hbm, v_hbm, uemb_hbm, vemb_hbm, ue_hbm, ve_hbm,
               uidx, vidx, idx_stage, rbu, rbv, sem):
    wid = lax.axis_index("s") * NC + lax.axis_index("c")
    sid = lax.axis_index("s")
    base = wid * BPW
    pltpu.sync_copy(u_hbm.at[wid], idx_stage.at[sid, pl.ds(0, BPW)])
    pltpu.sync_copy(v_hbm.at[wid], idx_stage.at[sid, pl.ds(BPW, BPW)])
    pltpu.sync_copy(idx_stage.at[sid, pl.ds(0, BPW)], uidx)
    pltpu.sync_copy(idx_stage.at[sid, pl.ds(BPW, BPW)], vidx)

    def chunk_body(ch, carry):
        off = ch * CH

        def issue(k, c):
            w = uidx[off + k]
            pltpu.async_copy(
                uemb_hbm.at[lax.shift_right_logical(w, 3), lax.bitwise_and(w, 7)],
                rbu.at[k], sem)
            w = vidx[off + k]
            pltpu.async_copy(
                vemb_hbm.at[lax.shift_right_logical(w, 3), lax.bitwise_and(w, 7)],
                rbv.at[k], sem)
            return c

        lax.fori_loop(0, CH, issue, 0)

        def drain(k, c):
            pltpu.make_async_copy(uemb_hbm.at[0, 0], rbu.at[0], sem).wait()
            pltpu.make_async_copy(vemb_hbm.at[0, 0], rbv.at[0], sem).wait()
            return c

        lax.fori_loop(0, CH, drain, 0)
        pltpu.sync_copy(rbu, ue_hbm.at[pl.ds(base + off, CH)])
        pltpu.sync_copy(rbv, ve_hbm.at[pl.ds(base + off, CH)])
        return carry

    lax.fori_loop(0, NCH, chunk_body, 0)


MLP_BLK = 2048


def _mlp_body(ue, ve, w1a, w1b, b1, w2, b2, w3, b3, w4, b4, w5, b5, out):
    x = jnp.maximum(ue[...] @ w1a[...] + ve[...] @ w1b[...] + b1[...], 0.0)
    x = jnp.maximum(x @ w2[...] + b2[...], 0.0)
    x = jnp.maximum(x @ w3[...] + b3[...], 0.0)
    x = jnp.maximum(x @ w4[...] + b4[...], 0.0)
    out[...] = x @ w5[...] + b5[...]


def _full(shape):
    return pl.BlockSpec(shape, lambda i: (0, 0))


_mlp_call = pl.pallas_call(
    _mlp_body,
    grid=(BATCH // MLP_BLK,),
    in_specs=[
        pl.BlockSpec((MLP_BLK, EMB), lambda i: (i, 0)),
        pl.BlockSpec((MLP_BLK, EMB), lambda i: (i, 0)),
        _full((EMB, 64)), _full((EMB, 64)), _full((1, 64)),
        _full((64, 32)), _full((1, 32)),
        _full((32, 16)), _full((1, 16)),
        _full((16, 8)), _full((1, 8)),
        _full((8, 1)), _full((1, 1)),
    ],
    out_specs=pl.BlockSpec((MLP_BLK, 1), lambda i: (i, 0)),
    out_shape=jax.ShapeDtypeStruct((BATCH, 1), jnp.float32),
)


def kernel(u, v, user_emb, item_emb, W1, b1, W2, b2, W3, b3, W4, b4, W5, b5):
    u = u.astype(jnp.int32).reshape(NW, BPW)
    v = v.astype(jnp.int32).reshape(NW, BPW)
    uemb = user_emb.reshape(NGRP, TILE_H, EMB)
    vemb = item_emb.reshape(NGRP, TILE_H, EMB)
    ue, ve = _sc_gather(u, v, uemb, vemb)
    out = _mlp_call(
        ue, ve,
        W1[:EMB], W1[EMB:], b1.reshape(1, 64),
        W2, b2.reshape(1, 32),
        W3, b3.reshape(1, 16),
        W4, b4.reshape(1, 8),
        W5, b5.reshape(1, 1),
    )
    return jnp.squeeze(out, axis=1)
